# baseline (speedup 1.0000x reference)
import jax
import jax.numpy as jnp
import numpy as np
from jax.experimental import pallas as pl
from jax.experimental.pallas import tpu as pltpu

_EMBED = 4096
_NE = 64
_K = 8
_NT = 16384
_BT = 1024

_EPS_CACHE = None


def _eps_const():
    global _EPS_CACHE
    if _EPS_CACHE is None:
        with jax.ensure_compile_time_eval():
            eps = jax.random.normal(jax.random.key(42), (_NT, _NE), dtype=jnp.float32)
            _EPS_CACHE = np.asarray(eps).T.copy()
    return _EPS_CACHE


def _router_block(x_ref, wr_ref, wn_ref, br_ref, bn_ref, epsT_ref,
                  router_ref, idx_ref):
    x = x_ref[...]
    logits = jnp.dot(x, wr_ref[...], preferred_element_type=jnp.float32)
    noise_logits = jnp.dot(x, wn_ref[...], preferred_element_type=jnp.float32)
    logitsT = (logits + br_ref[...]).T
    noiseT = (noise_logits + bn_ref[...]).T
    noisyT = logitsT + epsT_ref[...] * jax.nn.softplus(noiseT)

    _C = 128
    rowc = jax.lax.broadcasted_iota(jnp.int32, (_NE, _C), 0)
    for c in range(_BT // _C):
        noisy = noisyT[:, c * _C:(c + 1) * _C]
        work = noisy
        sel = jnp.zeros((_NE, _C), jnp.bool_)
        idx_rows = []
        m0 = None
        for k in range(_K):
            m = jnp.max(work, axis=0, keepdims=True)
            if k == 0:
                m0 = m
            idx = jnp.min(jnp.where(work == m, rowc, _NE), axis=0, keepdims=True)
            hit = rowc == idx
            sel = jnp.logical_or(sel, hit)
            work = jnp.where(hit, -jnp.inf, work)
            idx_rows.append(idx)

        p = jnp.where(sel, jnp.exp(noisy - m0), 0.0)
        denom = jnp.sum(p, axis=0, keepdims=True)
        router_ref[pl.ds(c * _C, _C), :] = (p / denom).T
        idx_ref[pl.ds(c * _C, _C), :] = jnp.concatenate(idx_rows, axis=0).T


def kernel(mh_output, W_route, b_route, W_noise, b_noise):
    epsT = jnp.asarray(_eps_const())
    router, idx = pl.pallas_call(
        _router_block,
        grid=(_NT // _BT,),
        in_specs=[
            pl.BlockSpec((_BT, _EMBED), lambda i: (i, 0)),
            pl.BlockSpec((_EMBED, _NE), lambda i: (0, 0)),
            pl.BlockSpec((_EMBED, _NE), lambda i: (0, 0)),
            pl.BlockSpec((1, _NE), lambda i: (0, 0)),
            pl.BlockSpec((1, _NE), lambda i: (0, 0)),
            pl.BlockSpec((_NE, _BT), lambda i: (0, i)),
        ],
        out_specs=[
            pl.BlockSpec((_BT, _NE), lambda i: (i, 0)),
            pl.BlockSpec((_BT, _K), lambda i: (i, 0)),
        ],
        out_shape=[
            jax.ShapeDtypeStruct((_NT, _NE), jnp.float32),
            jax.ShapeDtypeStruct((_NT, _K), jnp.int32),
        ],
        compiler_params=pltpu.CompilerParams(
            dimension_semantics=("parallel",),
        ),
    )(mh_output, W_route, W_noise, b_route[None, :], b_noise[None, :], epsT)
    return (router, idx)


# speedup vs baseline: 1.0985x; 1.0985x over previous
import jax
import jax.numpy as jnp
import numpy as np
from jax.experimental import pallas as pl
from jax.experimental.pallas import tpu as pltpu

_EMBED = 4096
_NE = 64
_K = 8
_NT = 16384
_BT = 1024

_EPS_CACHE = None


def _eps_const():
    global _EPS_CACHE
    if _EPS_CACHE is None:
        with jax.ensure_compile_time_eval():
            eps = jax.random.normal(jax.random.key(42), (_NT, _NE), dtype=jnp.float32)
            _EPS_CACHE = np.asarray(eps).T.copy()
    return _EPS_CACHE


def _router_block(xlo_ref, xhi_ref, w_ref, b_ref, epsT_ref, router_ref, idx_ref):
    acc = jnp.dot(xlo_ref[...], w_ref[: _EMBED // 2, :],
                  preferred_element_type=jnp.float32)
    acc = acc + jnp.dot(xhi_ref[...], w_ref[_EMBED // 2:, :],
                        preferred_element_type=jnp.float32)
    acc = acc + b_ref[...]
    accT = acc.T
    logitsT = accT[:_NE, :]
    noiseT = accT[_NE:, :]
    noisyT = logitsT + epsT_ref[...] * jax.nn.softplus(noiseT)

    _C = 128
    rowc = jax.lax.broadcasted_iota(jnp.int32, (_NE, _C), 0)
    for c in range(_BT // _C):
        noisy = noisyT[:, c * _C:(c + 1) * _C]
        work = noisy
        sel = jnp.zeros((_NE, _C), jnp.bool_)
        idx_rows = []
        m0 = None
        for k in range(_K):
            m = jnp.max(work, axis=0, keepdims=True)
            if k == 0:
                m0 = m
            idx = jnp.min(jnp.where(work == m, rowc, _NE), axis=0, keepdims=True)
            hit = rowc == idx
            sel = jnp.logical_or(sel, hit)
            work = jnp.where(hit, -jnp.inf, work)
            idx_rows.append(idx)

        p = jnp.where(sel, jnp.exp(noisy - m0), 0.0)
        denom = jnp.sum(p, axis=0, keepdims=True)
        router_ref[pl.ds(c * _C, _C), :] = (p / denom).T
        idx_ref[pl.ds(c * _C, _C), :] = jnp.concatenate(idx_rows, axis=0).T


def kernel(mh_output, W_route, b_route, W_noise, b_noise):
    epsT = jnp.asarray(_eps_const())
    w = jnp.concatenate([W_route, W_noise], axis=1)
    b = jnp.concatenate([b_route, b_noise])[None, :]
    router, idx = pl.pallas_call(
        _router_block,
        grid=(_NT // _BT,),
        in_specs=[
            pl.BlockSpec((_BT, _EMBED // 2), lambda i: (i, 0)),
            pl.BlockSpec((_BT, _EMBED // 2), lambda i: (i, 1)),
            pl.BlockSpec((_EMBED, 2 * _NE), lambda i: (0, 0)),
            pl.BlockSpec((1, 2 * _NE), lambda i: (0, 0)),
            pl.BlockSpec((_NE, _BT), lambda i: (0, i)),
        ],
        out_specs=[
            pl.BlockSpec((_BT, _NE), lambda i: (i, 0)),
            pl.BlockSpec((_BT, _K), lambda i: (i, 0)),
        ],
        out_shape=[
            jax.ShapeDtypeStruct((_NT, _NE), jnp.float32),
            jax.ShapeDtypeStruct((_NT, _K), jnp.int32),
        ],
        compiler_params=pltpu.CompilerParams(
            dimension_semantics=("parallel",),
        ),
    )(mh_output, mh_output, w, b, epsT)
    return (router, idx)


# speedup vs baseline: 1.1227x; 1.0220x over previous
import jax
import jax.numpy as jnp
import numpy as np
from jax.experimental import pallas as pl
from jax.experimental.pallas import tpu as pltpu

_EMBED = 4096
_NE = 64
_K = 8
_NT = 16384
_BT = 1024

_EPS_CACHE = None


def _eps_const():
    global _EPS_CACHE
    if _EPS_CACHE is None:
        with jax.ensure_compile_time_eval():
            eps = jax.random.normal(jax.random.key(42), (_NT, _NE), dtype=jnp.float32)
            _EPS_CACHE = np.asarray(eps).T.copy()
    return _EPS_CACHE


def _router_block(x_ref, w_ref, b_ref, epsT_ref, router_ref, idx_ref):
    acc = jnp.dot(x_ref[...], w_ref[...], preferred_element_type=jnp.float32)
    acc = acc + b_ref[...]
    accT = acc.T
    logitsT = accT[:_NE, :]
    noiseT = accT[_NE:, :]
    noisyT = logitsT + epsT_ref[...] * jax.nn.softplus(noiseT)

    _C = 128
    rowc = jax.lax.broadcasted_iota(jnp.int32, (_NE, _C), 0)
    for c in range(_BT // _C):
        noisy = noisyT[:, c * _C:(c + 1) * _C]
        work = noisy
        sel = jnp.zeros((_NE, _C), jnp.bool_)
        idx_rows = []
        m0 = None
        for k in range(_K):
            m = jnp.max(work, axis=0, keepdims=True)
            if k == 0:
                m0 = m
            idx = jnp.min(jnp.where(work == m, rowc, _NE), axis=0, keepdims=True)
            hit = rowc == idx
            sel = jnp.logical_or(sel, hit)
            work = jnp.where(hit, -jnp.inf, work)
            idx_rows.append(idx)

        p = jnp.where(sel, jnp.exp(noisy - m0), 0.0)
        denom = jnp.sum(p, axis=0, keepdims=True)
        router_ref[pl.ds(c * _C, _C), :] = (p / denom).T
        idx_ref[pl.ds(c * _C, _C), :] = jnp.concatenate(idx_rows, axis=0).T


def kernel(mh_output, W_route, b_route, W_noise, b_noise):
    epsT = jnp.asarray(_eps_const())
    w = jnp.concatenate([W_route, W_noise], axis=1)
    b = jnp.concatenate([b_route, b_noise])[None, :]
    router, idx = pl.pallas_call(
        _router_block,
        grid=(_NT // _BT,),
        in_specs=[
            pl.BlockSpec((_BT, _EMBED), lambda i: (i, 0)),
            pl.BlockSpec((_EMBED, 2 * _NE), lambda i: (0, 0)),
            pl.BlockSpec((1, 2 * _NE), lambda i: (0, 0)),
            pl.BlockSpec((_NE, _BT), lambda i: (0, i)),
        ],
        out_specs=[
            pl.BlockSpec((_BT, _NE), lambda i: (i, 0)),
            pl.BlockSpec((_BT, _K), lambda i: (i, 0)),
        ],
        out_shape=[
            jax.ShapeDtypeStruct((_NT, _NE), jnp.float32),
            jax.ShapeDtypeStruct((_NT, _K), jnp.int32),
        ],
        compiler_params=pltpu.CompilerParams(
            dimension_semantics=("parallel",),
        ),
    )(mh_output, w, b, epsT)
    return (router, idx)
